# Initial kernel scaffold; baseline (speedup 1.0000x reference)
import jax
import jax.numpy as jnp
from jax.experimental import pallas as pl


def kernel(users, items, edge_index, user_emb, item_emb):
    raise NotImplementedError("write your pallas kernel here")


# speedup vs baseline: 7.1827x; 7.1827x over previous
import functools

import jax
import jax.numpy as jnp
from jax import lax
from jax.experimental import pallas as pl
from jax.experimental.pallas import tpu as pltpu
from jax.experimental.pallas import tpu_sc as plsc

NU = 5000
NI = 5000
D = 128
B = 4096
NP = 5120
PAD_ROW = NP - 1
NSUB = 16
CHUNK = 128
RB = 64

f32 = jnp.float32
i32 = jnp.int32

_mesh = plsc.VectorSubcoreMesh(core_axis_name="c", subcore_axis_name="s")


def _sds(shape, dtype):
    return jax.ShapeDtypeStruct(shape, dtype)


def _deg_body(cps, uidx_hbm, iidx_hbm, ones_hbm, zrows_hbm,
              degu_hbm, degi_hbm, idx_v, ones_v, sem, acc_sh):
    core = lax.axis_index("c")
    sid = lax.axis_index("s")
    rps = NP // NSUB

    @pl.loop(0, rps // RB)
    def _(j):
        pltpu.sync_copy(zrows_hbm, acc_sh.at[pl.ds(sid * rps + j * RB, RB)])

    pltpu.sync_copy(ones_hbm, ones_v)
    plsc.subcore_barrier()

    def _count(idx2d_hbm, deg_hbm):
        pltpu.sync_copy(idx2d_hbm.at[pl.ds(sid * cps, cps)], idx_v)

        @pl.loop(0, cps)
        def _(k):
            pltpu.sync_copy(ones_v, acc_sh.at[idx_v.at[k]], add=True)

        plsc.subcore_barrier()
        pltpu.sync_copy(acc_sh.at[pl.ds(sid * rps, rps)],
                        deg_hbm.at[pl.ds(sid * rps, rps)])

    @pl.when(core == 0)
    def _():
        _count(uidx_hbm, degu_hbm)

    @pl.when(core == 1)
    def _():
        _count(iidx_hbm, degi_hbm)


def _deg(u2d, i2d, onesD, zrows):
    cps = u2d.shape[0] // NSUB
    body = functools.partial(_deg_body, cps)
    k = pl.kernel(
        body,
        mesh=_mesh,
        out_type=[_sds((NP, D), f32), _sds((NP, D), f32)],
        scratch_types=[
            pltpu.VMEM((cps, CHUNK), i32),
            pltpu.VMEM((CHUNK, D), f32),
            pltpu.SemaphoreType.DMA,
            pltpu.VMEM_SHARED((NP, D), f32),
        ],
    )
    return k(u2d, i2d, onesD, zrows)


def _layer_body(cps, uidx_hbm, iidx_hbm, yu_hbm, yi_hbm, dinvu_hbm, dinvi_hbm,
                zrows_hbm, you_hbm, yoi_hbm,
                idxg_v, idxs_v, rows_v, stage_v, dinv_v, sem, acc_sh):
    core = lax.axis_index("c")
    sid = lax.axis_index("s")
    rps = NP // NSUB

    @pl.loop(0, rps // RB)
    def _(j):
        pltpu.sync_copy(zrows_hbm, acc_sh.at[pl.ds(sid * rps + j * RB, RB)])

    plsc.subcore_barrier()

    def _side(idxg2d_hbm, idxs2d_hbm, ysrc_hbm, dinv_hbm, yout_hbm):
        pltpu.sync_copy(idxg2d_hbm.at[pl.ds(sid * cps, cps)], idxg_v)
        pltpu.sync_copy(idxs2d_hbm.at[pl.ds(sid * cps, cps)], idxs_v)

        @pl.loop(0, cps)
        def _(k):
            pltpu.sync_copy(ysrc_hbm.at[idxg_v.at[k]], rows_v)
            pltpu.sync_copy(rows_v, acc_sh.at[idxs_v.at[k]], add=True)

        plsc.subcore_barrier()

        @pl.loop(0, rps // RB)
        def _(j):
            r0 = sid * rps + j * RB
            pltpu.sync_copy(acc_sh.at[pl.ds(r0, RB)], stage_v)
            pltpu.sync_copy(dinv_hbm.at[pl.ds(r0, RB)], dinv_v)

            @pl.loop(0, RB)
            def _(r):
                dv = dinv_v[r, :]
                for q in range(D // 16):
                    sl = pl.ds(q * 16, 16)
                    stage_v[r, sl] = stage_v[r, sl] * dv

            pltpu.sync_copy(stage_v, yout_hbm.at[pl.ds(r0, RB)])

    @pl.when(core == 0)
    def _():
        _side(iidx_hbm, uidx_hbm, yi_hbm, dinvu_hbm, you_hbm)

    @pl.when(core == 1)
    def _():
        _side(uidx_hbm, iidx_hbm, yu_hbm, dinvi_hbm, yoi_hbm)


def _layer(u2d, i2d, yu, yi, dinvu, dinvi, zrows):
    cps = u2d.shape[0] // NSUB
    body = functools.partial(_layer_body, cps)
    k = pl.kernel(
        body,
        mesh=_mesh,
        out_type=[_sds((NP, D), f32), _sds((NP, D), f32)],
        scratch_types=[
            pltpu.VMEM((cps, CHUNK), i32),
            pltpu.VMEM((cps, CHUNK), i32),
            pltpu.VMEM((CHUNK, D), f32),
            pltpu.VMEM((RB, D), f32),
            pltpu.VMEM((RB, 16), f32),
            pltpu.SemaphoreType.DMA,
            pltpu.VMEM_SHARED((NP, D), f32),
        ],
    )
    return k(u2d, i2d, yu, yi, dinvu, dinvi, zrows)


def _bgather_body(rows_per_sub, zu_hbm, zi_hbm, users_hbm, items_hbm,
                  gu_hbm, gi_hbm, idx_v, rows_v, sem):
    core = lax.axis_index("c")
    sid = lax.axis_index("s")
    nck = rows_per_sub // CHUNK

    def _side(idx2d_hbm, z_hbm, g_hbm):
        pltpu.sync_copy(idx2d_hbm, idx_v)

        @pl.loop(0, nck)
        def _(k):
            pltpu.sync_copy(z_hbm.at[idx_v.at[sid * nck + k]], rows_v)
            pltpu.sync_copy(rows_v,
                            g_hbm.at[pl.ds((sid * nck + k) * CHUNK, CHUNK)])

    @pl.when(core == 0)
    def _():
        _side(users_hbm, zu_hbm, gu_hbm)

    @pl.when(core == 1)
    def _():
        _side(items_hbm, zi_hbm, gi_hbm)


def _bgather(zu, zi, users2d, items2d):
    rows_per_sub = B // NSUB
    body = functools.partial(_bgather_body, rows_per_sub)
    k = pl.kernel(
        body,
        mesh=_mesh,
        out_type=[_sds((B, D), f32), _sds((B, D), f32)],
        scratch_types=[
            pltpu.VMEM((B // CHUNK, CHUNK), i32),
            pltpu.VMEM((CHUNK, D), f32),
            pltpu.SemaphoreType.DMA,
        ],
    )
    return k(zu, zi, users2d, items2d)


def _prep_tc_body(ue_ref, ie_ref, degu_ref, degi_ref,
                  y0u_ref, y0i_ref, dinvu_ref, dinvi_ref):
    du = jnp.maximum(degu_ref[:, 0:1], 1.0)
    di = jnp.maximum(degi_ref[:, 0:1], 1.0)
    y0u_ref[...] = ue_ref[...] * lax.rsqrt(du)
    y0i_ref[...] = ie_ref[...] * lax.rsqrt(di)
    dinvu_ref[...] = jnp.broadcast_to(1.0 / du, (NP, 16))
    dinvi_ref[...] = jnp.broadcast_to(1.0 / di, (NP, 16))


def _prep(uep, iep, degu, degi):
    return pl.pallas_call(
        _prep_tc_body,
        out_shape=[_sds((NP, D), f32), _sds((NP, D), f32),
                   _sds((NP, 16), f32), _sds((NP, 16), f32)],
    )(uep, iep, degu, degi)


def _comb_tc_body(y0u, y1u, y2u, y3u, y0i, y1i, y2i, y3i, degu, degi,
                  zu_ref, zi_ref):
    su = jnp.sqrt(jnp.maximum(degu[:, 0:1], 1.0)) * 0.25
    si = jnp.sqrt(jnp.maximum(degi[:, 0:1], 1.0)) * 0.25
    zu_ref[...] = (y0u[...] + y1u[...] + y2u[...] + y3u[...]) * su
    zi_ref[...] = (y0i[...] + y1i[...] + y2i[...] + y3i[...]) * si


def _comb(y0u, y1u, y2u, y3u, y0i, y1i, y2i, y3i, degu, degi):
    return pl.pallas_call(
        _comb_tc_body,
        out_shape=[_sds((NP, D), f32), _sds((NP, D), f32)],
    )(y0u, y1u, y2u, y3u, y0i, y1i, y2i, y3i, degu, degi)


def _gamma_tc_body(gu_ref, gi_ref, out_ref):
    out_ref[...] = jnp.sum(gu_ref[...] * gi_ref[...], axis=1, keepdims=True)


def _gamma(gu, gi):
    return pl.pallas_call(
        _gamma_tc_body,
        out_shape=_sds((B, 1), f32),
    )(gu, gi)


def kernel(users, items, edge_index, user_emb, item_emb):
    e2 = edge_index.shape[1] // 2
    e2p = -(-e2 // (NSUB * CHUNK * 8)) * (NSUB * CHUNK * 8)

    u = edge_index[0, :e2]
    it = edge_index[1, :e2] - NU
    pad = jnp.full((e2p - e2,), PAD_ROW, i32)
    u2d = jnp.concatenate([u, pad]).reshape(e2p // CHUNK, CHUNK)
    i2d = jnp.concatenate([it, pad]).reshape(e2p // CHUNK, CHUNK)

    onesD = jnp.ones((CHUNK, D), f32)
    zrows = jnp.zeros((RB, D), f32)
    uep = jnp.pad(user_emb, ((0, NP - NU), (0, 0)))
    iep = jnp.pad(item_emb, ((0, NP - NI), (0, 0)))
    users2d = users.reshape(B // CHUNK, CHUNK)
    items2d = items.reshape(B // CHUNK, CHUNK)

    degu, degi = _deg(u2d, i2d, onesD, zrows)
    y0u, y0i, dinvu, dinvi = _prep(uep, iep, degu, degi)
    y1u, y1i = _layer(u2d, i2d, y0u, y0i, dinvu, dinvi, zrows)
    y2u, y2i = _layer(u2d, i2d, y1u, y1i, dinvu, dinvi, zrows)
    y3u, y3i = _layer(u2d, i2d, y2u, y2i, dinvu, dinvi, zrows)
    zu, zi = _comb(y0u, y1u, y2u, y3u, y0i, y1i, y2i, y3i, degu, degi)
    gu, gi = _bgather(zu, zi, users2d, items2d)
    return _gamma(gu, gi)[:, 0]


# speedup vs baseline: 8.4801x; 1.1806x over previous
import functools

import jax
import jax.numpy as jnp
from jax import lax
from jax.experimental import pallas as pl
from jax.experimental.pallas import tpu as pltpu
from jax.experimental.pallas import tpu_sc as plsc

NU = 5000
NI = 5000
D = 128
B = 4096
NP = 5120
PAD_ROW = NP - 1
NSUB = 16
CHUNK = 128
RB = 64

f32 = jnp.float32
i32 = jnp.int32

_mesh = plsc.VectorSubcoreMesh(core_axis_name="c", subcore_axis_name="s")


def _sds(shape, dtype):
    return jax.ShapeDtypeStruct(shape, dtype)


def _deg_body(cps, uidx_hbm, iidx_hbm, ones_hbm, zrows_hbm,
              degu_hbm, degi_hbm, idx_v, ones_v, sem, acc_sh):
    core = lax.axis_index("c")
    sid = lax.axis_index("s")
    rps = NP // NSUB

    @pl.loop(0, rps // RB)
    def _(j):
        pltpu.sync_copy(zrows_hbm, acc_sh.at[pl.ds(sid * rps + j * RB, RB)])

    pltpu.sync_copy(ones_hbm, ones_v)
    plsc.subcore_barrier()

    def _count(idx2d_hbm, deg_hbm):
        pltpu.sync_copy(idx2d_hbm.at[pl.ds(sid * cps, cps)], idx_v)

        @pl.loop(0, cps)
        def _(k):
            pltpu.sync_copy(ones_v, acc_sh.at[idx_v.at[k]], add=True)

        plsc.subcore_barrier()
        pltpu.sync_copy(acc_sh.at[pl.ds(sid * rps, rps)],
                        deg_hbm.at[pl.ds(sid * rps, rps)])

    @pl.when(core == 0)
    def _():
        _count(uidx_hbm, degu_hbm)

    @pl.when(core == 1)
    def _():
        _count(iidx_hbm, degi_hbm)


def _deg(u2d, i2d, onesD, zrows):
    cps = u2d.shape[0] // NSUB
    body = functools.partial(_deg_body, cps)
    k = pl.kernel(
        body,
        mesh=_mesh,
        out_type=[_sds((NP, D), f32), _sds((NP, D), f32)],
        scratch_types=[
            pltpu.VMEM((cps, CHUNK), i32),
            pltpu.VMEM((CHUNK, D), f32),
            pltpu.SemaphoreType.DMA,
            pltpu.VMEM_SHARED((NP, D), f32),
        ],
    )
    return k(u2d, i2d, onesD, zrows)


def _layer_body(cps, uidx_hbm, iidx_hbm, yu_hbm, yi_hbm, dinvu_hbm, dinvi_hbm,
                zrows_hbm, you_hbm, yoi_hbm,
                idxg_v, idxs_v, rows_a, rows_b, stage_v, dinv_v,
                sem_a, sem_b, acc_sh):
    core = lax.axis_index("c")
    sid = lax.axis_index("s")
    rps = NP // NSUB

    @pl.loop(0, rps // RB)
    def _(j):
        pltpu.sync_copy(zrows_hbm, acc_sh.at[pl.ds(sid * rps + j * RB, RB)])

    plsc.subcore_barrier()

    def _side(idxg2d_hbm, idxs2d_hbm, ysrc_hbm, dinv_hbm, yout_hbm):
        pltpu.sync_copy(idxg2d_hbm.at[pl.ds(sid * cps, cps)], idxg_v)
        pltpu.sync_copy(idxs2d_hbm.at[pl.ds(sid * cps, cps)], idxs_v)

        pltpu.async_copy(ysrc_hbm.at[idxg_v.at[0]], rows_a, sem_a)

        @pl.loop(0, cps, step=2)
        def _(k):
            pltpu.async_copy(ysrc_hbm.at[idxg_v.at[k + 1]], rows_b, sem_b)
            pltpu.make_async_copy(ysrc_hbm.at[idxg_v.at[k]], rows_a,
                                  sem_a).wait()
            pltpu.sync_copy(rows_a, acc_sh.at[idxs_v.at[k]], add=True)

            @pl.when(k + 2 < cps)
            def _():
                pltpu.async_copy(ysrc_hbm.at[idxg_v.at[k + 2]], rows_a, sem_a)

            pltpu.make_async_copy(ysrc_hbm.at[idxg_v.at[k + 1]], rows_b,
                                  sem_b).wait()
            pltpu.sync_copy(rows_b, acc_sh.at[idxs_v.at[k + 1]], add=True)

        plsc.subcore_barrier()

        @pl.loop(0, rps // RB)
        def _(j):
            r0 = sid * rps + j * RB
            pltpu.sync_copy(acc_sh.at[pl.ds(r0, RB)], stage_v)
            pltpu.sync_copy(dinv_hbm.at[pl.ds(r0, RB)], dinv_v)

            @pl.loop(0, RB)
            def _(r):
                dv = dinv_v[r, :]
                for q in range(D // 16):
                    sl = pl.ds(q * 16, 16)
                    stage_v[r, sl] = stage_v[r, sl] * dv

            pltpu.sync_copy(stage_v, yout_hbm.at[pl.ds(r0, RB)])

    @pl.when(core == 0)
    def _():
        _side(iidx_hbm, uidx_hbm, yi_hbm, dinvu_hbm, you_hbm)

    @pl.when(core == 1)
    def _():
        _side(uidx_hbm, iidx_hbm, yu_hbm, dinvi_hbm, yoi_hbm)


def _layer(u2d, i2d, yu, yi, dinvu, dinvi, zrows):
    cps = u2d.shape[0] // NSUB
    body = functools.partial(_layer_body, cps)
    k = pl.kernel(
        body,
        mesh=_mesh,
        out_type=[_sds((NP, D), f32), _sds((NP, D), f32)],
        scratch_types=[
            pltpu.VMEM((cps, CHUNK), i32),
            pltpu.VMEM((cps, CHUNK), i32),
            pltpu.VMEM((CHUNK, D), f32),
            pltpu.VMEM((CHUNK, D), f32),
            pltpu.VMEM((RB, D), f32),
            pltpu.VMEM((RB, 16), f32),
            pltpu.SemaphoreType.DMA,
            pltpu.SemaphoreType.DMA,
            pltpu.VMEM_SHARED((NP, D), f32),
        ],
    )
    return k(u2d, i2d, yu, yi, dinvu, dinvi, zrows)


def _bgather_body(rows_per_sub, zu_hbm, zi_hbm, users_hbm, items_hbm,
                  gu_hbm, gi_hbm, idx_v, rows_v, sem):
    core = lax.axis_index("c")
    sid = lax.axis_index("s")
    nck = rows_per_sub // CHUNK

    def _side(idx2d_hbm, z_hbm, g_hbm):
        pltpu.sync_copy(idx2d_hbm, idx_v)

        @pl.loop(0, nck)
        def _(k):
            pltpu.sync_copy(z_hbm.at[idx_v.at[sid * nck + k]], rows_v)
            pltpu.sync_copy(rows_v,
                            g_hbm.at[pl.ds((sid * nck + k) * CHUNK, CHUNK)])

    @pl.when(core == 0)
    def _():
        _side(users_hbm, zu_hbm, gu_hbm)

    @pl.when(core == 1)
    def _():
        _side(items_hbm, zi_hbm, gi_hbm)


def _bgather(zu, zi, users2d, items2d):
    rows_per_sub = B // NSUB
    body = functools.partial(_bgather_body, rows_per_sub)
    k = pl.kernel(
        body,
        mesh=_mesh,
        out_type=[_sds((B, D), f32), _sds((B, D), f32)],
        scratch_types=[
            pltpu.VMEM((B // CHUNK, CHUNK), i32),
            pltpu.VMEM((CHUNK, D), f32),
            pltpu.SemaphoreType.DMA,
        ],
    )
    return k(zu, zi, users2d, items2d)


def _prep_tc_body(ue_ref, ie_ref, degu_ref, degi_ref,
                  y0u_ref, y0i_ref, dinvu_ref, dinvi_ref):
    du = jnp.maximum(degu_ref[:, 0:1], 1.0)
    di = jnp.maximum(degi_ref[:, 0:1], 1.0)
    y0u_ref[...] = ue_ref[...] * lax.rsqrt(du)
    y0i_ref[...] = ie_ref[...] * lax.rsqrt(di)
    dinvu_ref[...] = jnp.broadcast_to(1.0 / du, (NP, 16))
    dinvi_ref[...] = jnp.broadcast_to(1.0 / di, (NP, 16))


def _prep(uep, iep, degu, degi):
    return pl.pallas_call(
        _prep_tc_body,
        out_shape=[_sds((NP, D), f32), _sds((NP, D), f32),
                   _sds((NP, 16), f32), _sds((NP, 16), f32)],
    )(uep, iep, degu, degi)


def _comb_tc_body(y0u, y1u, y2u, y3u, y0i, y1i, y2i, y3i, degu, degi,
                  zu_ref, zi_ref):
    su = jnp.sqrt(jnp.maximum(degu[:, 0:1], 1.0)) * 0.25
    si = jnp.sqrt(jnp.maximum(degi[:, 0:1], 1.0)) * 0.25
    zu_ref[...] = (y0u[...] + y1u[...] + y2u[...] + y3u[...]) * su
    zi_ref[...] = (y0i[...] + y1i[...] + y2i[...] + y3i[...]) * si


def _comb(y0u, y1u, y2u, y3u, y0i, y1i, y2i, y3i, degu, degi):
    return pl.pallas_call(
        _comb_tc_body,
        out_shape=[_sds((NP, D), f32), _sds((NP, D), f32)],
    )(y0u, y1u, y2u, y3u, y0i, y1i, y2i, y3i, degu, degi)


def _gamma_tc_body(gu_ref, gi_ref, out_ref):
    out_ref[...] = jnp.sum(gu_ref[...] * gi_ref[...], axis=1, keepdims=True)


def _gamma(gu, gi):
    return pl.pallas_call(
        _gamma_tc_body,
        out_shape=_sds((B, 1), f32),
    )(gu, gi)


def kernel(users, items, edge_index, user_emb, item_emb):
    e2 = edge_index.shape[1] // 2
    e2p = -(-e2 // (NSUB * CHUNK * 8)) * (NSUB * CHUNK * 8)

    u = edge_index[0, :e2]
    it = edge_index[1, :e2] - NU
    pad = jnp.full((e2p - e2,), PAD_ROW, i32)
    u2d = jnp.concatenate([u, pad]).reshape(e2p // CHUNK, CHUNK)
    i2d = jnp.concatenate([it, pad]).reshape(e2p // CHUNK, CHUNK)

    onesD = jnp.ones((CHUNK, D), f32)
    zrows = jnp.zeros((RB, D), f32)
    uep = jnp.pad(user_emb, ((0, NP - NU), (0, 0)))
    iep = jnp.pad(item_emb, ((0, NP - NI), (0, 0)))
    users2d = users.reshape(B // CHUNK, CHUNK)
    items2d = items.reshape(B // CHUNK, CHUNK)

    degu, degi = _deg(u2d, i2d, onesD, zrows)
    y0u, y0i, dinvu, dinvi = _prep(uep, iep, degu, degi)
    y1u, y1i = _layer(u2d, i2d, y0u, y0i, dinvu, dinvi, zrows)
    y2u, y2i = _layer(u2d, i2d, y1u, y1i, dinvu, dinvi, zrows)
    y3u, y3i = _layer(u2d, i2d, y2u, y2i, dinvu, dinvi, zrows)
    zu, zi = _comb(y0u, y1u, y2u, y3u, y0i, y1i, y2i, y3i, degu, degi)
    gu, gi = _bgather(zu, zi, users2d, items2d)
    return _gamma(gu, gi)[:, 0]


# speedup vs baseline: 13.5407x; 1.5968x over previous
import functools

import jax
import jax.numpy as jnp
from jax import lax
from jax.experimental import pallas as pl
from jax.experimental.pallas import tpu as pltpu
from jax.experimental.pallas import tpu_sc as plsc

NU = 5000
NI = 5000
D = 128
B = 4096
NP = 5120
PAD_ROW = NP - 1
NSUB = 16
CHUNK = 128
RB = 64
NBUF = 3

f32 = jnp.float32
i32 = jnp.int32

_mesh = plsc.VectorSubcoreMesh(core_axis_name="c", subcore_axis_name="s")


def _sds(shape, dtype):
    return jax.ShapeDtypeStruct(shape, dtype)


def _deg_body(cps, uidx_hbm, iidx_hbm, ones_hbm, zrows_hbm,
              degu_hbm, degi_hbm, idx_v, ones_v, sem, acc_sh):
    core = lax.axis_index("c")
    sid = lax.axis_index("s")
    rps = NP // NSUB

    @pl.loop(0, rps // RB)
    def _(j):
        pltpu.sync_copy(zrows_hbm, acc_sh.at[pl.ds(sid * rps + j * RB, RB)])

    pltpu.sync_copy(ones_hbm, ones_v)
    plsc.subcore_barrier()

    def _count(idx2d_hbm, deg_hbm):
        pltpu.sync_copy(idx2d_hbm.at[pl.ds(sid * cps, cps)], idx_v)

        @pl.loop(0, cps)
        def _(k):
            pltpu.sync_copy(ones_v, acc_sh.at[idx_v.at[k]], add=True)

        plsc.subcore_barrier()
        pltpu.sync_copy(acc_sh.at[pl.ds(sid * rps, rps)],
                        deg_hbm.at[pl.ds(sid * rps, rps)])

    @pl.when(core == 0)
    def _():
        _count(uidx_hbm, degu_hbm)

    @pl.when(core == 1)
    def _():
        _count(iidx_hbm, degi_hbm)


def _deg(u2d, i2d, onesD, zrows):
    cps = u2d.shape[0] // NSUB
    body = functools.partial(_deg_body, cps)
    k = pl.kernel(
        body,
        mesh=_mesh,
        out_type=[_sds((NP, D), f32), _sds((NP, D), f32)],
        scratch_types=[
            pltpu.VMEM((cps, CHUNK), i32),
            pltpu.VMEM((CHUNK, D), f32),
            pltpu.SemaphoreType.DMA,
            pltpu.VMEM_SHARED((NP, D), f32),
        ],
    )
    return k(u2d, i2d, onesD, zrows)


def _layer_body(cps, uidx_hbm, iidx_hbm, yu_hbm, yi_hbm, dinvu_hbm, dinvi_hbm,
                zrows_hbm, you_hbm, yoi_hbm,
                idxg_v, idxs_v, rows_a, rows_b, dinv_v, sem_a, sem_b,
                acc_sh, ysrc_sh):
    core = lax.axis_index("c")
    sid = lax.axis_index("s")
    rps = NP // NSUB
    hc = cps // 5

    @pl.loop(0, rps // RB)
    def _(j):
        pltpu.sync_copy(zrows_hbm, acc_sh.at[pl.ds(sid * rps + j * RB, RB)])

    @pl.when(core == 0)
    def _():
        pltpu.sync_copy(yi_hbm.at[pl.ds(sid * rps, rps)],
                        ysrc_sh.at[pl.ds(sid * rps, rps)])

    @pl.when(core == 1)
    def _():
        pltpu.sync_copy(yu_hbm.at[pl.ds(sid * rps, rps)],
                        ysrc_sh.at[pl.ds(sid * rps, rps)])

    plsc.subcore_barrier()

    def _side(idxg2d_hbm, idxs2d_hbm, dinv_hbm, yout_hbm):
        for h in range(5):
            base = sid * cps + h * hc
            pltpu.sync_copy(idxg2d_hbm.at[pl.ds(base, hc)], idxg_v)
            pltpu.sync_copy(idxs2d_hbm.at[pl.ds(base, hc)], idxs_v)

            pltpu.async_copy(ysrc_sh.at[idxg_v.at[0]], rows_a, sem_a)
            pltpu.async_copy(ysrc_sh.at[idxg_v.at[1]], rows_b, sem_b)

            @pl.loop(0, hc - 2, step=2)
            def _(k):
                pltpu.make_async_copy(ysrc_sh.at[idxg_v.at[k]], rows_a,
                                      sem_a).wait()
                pltpu.sync_copy(rows_a, acc_sh.at[idxs_v.at[k]], add=True)
                pltpu.async_copy(ysrc_sh.at[idxg_v.at[k + 2]], rows_a, sem_a)
                pltpu.make_async_copy(ysrc_sh.at[idxg_v.at[k + 1]], rows_b,
                                      sem_b).wait()
                pltpu.sync_copy(rows_b, acc_sh.at[idxs_v.at[k + 1]],
                                add=True)
                pltpu.async_copy(ysrc_sh.at[idxg_v.at[k + 3]], rows_b, sem_b)

            pltpu.make_async_copy(ysrc_sh.at[idxg_v.at[hc - 2]], rows_a,
                                  sem_a).wait()
            pltpu.sync_copy(rows_a, acc_sh.at[idxs_v.at[hc - 2]], add=True)
            pltpu.make_async_copy(ysrc_sh.at[idxg_v.at[hc - 1]], rows_b,
                                  sem_b).wait()
            pltpu.sync_copy(rows_b, acc_sh.at[idxs_v.at[hc - 1]], add=True)

        plsc.subcore_barrier()

        @pl.loop(0, rps // RB)
        def _(j):
            r0 = sid * rps + j * RB
            pltpu.sync_copy(acc_sh.at[pl.ds(r0, RB)], rows_a.at[pl.ds(0, RB)])
            pltpu.sync_copy(dinv_hbm.at[pl.ds(r0, RB)], dinv_v)

            @pl.loop(0, RB)
            def _(r):
                dv = dinv_v[r, :]
                for q in range(D // 16):
                    sl = pl.ds(q * 16, 16)
                    rows_a[r, sl] = rows_a[r, sl] * dv

            pltpu.sync_copy(rows_a.at[pl.ds(0, RB)], yout_hbm.at[pl.ds(r0, RB)])

    @pl.when(core == 0)
    def _():
        _side(iidx_hbm, uidx_hbm, dinvu_hbm, you_hbm)

    @pl.when(core == 1)
    def _():
        _side(uidx_hbm, iidx_hbm, dinvi_hbm, yoi_hbm)


def _layer(u2d, i2d, yu, yi, dinvu, dinvi, zrows):
    cps = u2d.shape[0] // NSUB
    body = functools.partial(_layer_body, cps)
    k = pl.kernel(
        body,
        mesh=_mesh,
        out_type=[_sds((NP, D), f32), _sds((NP, D), f32)],
        scratch_types=[
            pltpu.VMEM((cps // 5, CHUNK), i32),
            pltpu.VMEM((cps // 5, CHUNK), i32),
            pltpu.VMEM((CHUNK, D), f32),
            pltpu.VMEM((CHUNK, D), f32),
            pltpu.VMEM((RB, 16), f32),
            pltpu.SemaphoreType.DMA,
            pltpu.SemaphoreType.DMA,
            pltpu.VMEM_SHARED((NP, D), f32),
            pltpu.VMEM_SHARED((NP, D), f32),
        ],
    )
    return k(u2d, i2d, yu, yi, dinvu, dinvi, zrows)


def _bgather_body(rows_per_sub, zu_hbm, zi_hbm, users_hbm, items_hbm,
                  gu_hbm, gi_hbm, idx_v, rows_v, sem):
    core = lax.axis_index("c")
    sid = lax.axis_index("s")
    nck = rows_per_sub // CHUNK

    def _side(idx2d_hbm, z_hbm, g_hbm):
        pltpu.sync_copy(idx2d_hbm, idx_v)

        @pl.loop(0, nck)
        def _(k):
            pltpu.sync_copy(z_hbm.at[idx_v.at[sid * nck + k]], rows_v)
            pltpu.sync_copy(rows_v,
                            g_hbm.at[pl.ds((sid * nck + k) * CHUNK, CHUNK)])

    @pl.when(core == 0)
    def _():
        _side(users_hbm, zu_hbm, gu_hbm)

    @pl.when(core == 1)
    def _():
        _side(items_hbm, zi_hbm, gi_hbm)


def _bgather(zu, zi, users2d, items2d):
    rows_per_sub = B // NSUB
    body = functools.partial(_bgather_body, rows_per_sub)
    k = pl.kernel(
        body,
        mesh=_mesh,
        out_type=[_sds((B, D), f32), _sds((B, D), f32)],
        scratch_types=[
            pltpu.VMEM((B // CHUNK, CHUNK), i32),
            pltpu.VMEM((CHUNK, D), f32),
            pltpu.SemaphoreType.DMA,
        ],
    )
    return k(zu, zi, users2d, items2d)


def _prep_tc_body(ue_ref, ie_ref, degu_ref, degi_ref,
                  y0u_ref, y0i_ref, dinvu_ref, dinvi_ref):
    du = jnp.maximum(degu_ref[:, 0:1], 1.0)
    di = jnp.maximum(degi_ref[:, 0:1], 1.0)
    y0u_ref[...] = ue_ref[...] * lax.rsqrt(du)
    y0i_ref[...] = ie_ref[...] * lax.rsqrt(di)
    dinvu_ref[...] = jnp.broadcast_to(1.0 / du, (NP, 16))
    dinvi_ref[...] = jnp.broadcast_to(1.0 / di, (NP, 16))


def _prep(uep, iep, degu, degi):
    return pl.pallas_call(
        _prep_tc_body,
        out_shape=[_sds((NP, D), f32), _sds((NP, D), f32),
                   _sds((NP, 16), f32), _sds((NP, 16), f32)],
    )(uep, iep, degu, degi)


def _comb_tc_body(y0u, y1u, y2u, y3u, y0i, y1i, y2i, y3i, degu, degi,
                  zu_ref, zi_ref):
    su = jnp.sqrt(jnp.maximum(degu[:, 0:1], 1.0)) * 0.25
    si = jnp.sqrt(jnp.maximum(degi[:, 0:1], 1.0)) * 0.25
    zu_ref[...] = (y0u[...] + y1u[...] + y2u[...] + y3u[...]) * su
    zi_ref[...] = (y0i[...] + y1i[...] + y2i[...] + y3i[...]) * si


def _comb(y0u, y1u, y2u, y3u, y0i, y1i, y2i, y3i, degu, degi):
    return pl.pallas_call(
        _comb_tc_body,
        out_shape=[_sds((NP, D), f32), _sds((NP, D), f32)],
    )(y0u, y1u, y2u, y3u, y0i, y1i, y2i, y3i, degu, degi)


def _gamma_tc_body(gu_ref, gi_ref, out_ref):
    out_ref[...] = jnp.sum(gu_ref[...] * gi_ref[...], axis=1, keepdims=True)


def _gamma(gu, gi):
    return pl.pallas_call(
        _gamma_tc_body,
        out_shape=_sds((B, 1), f32),
    )(gu, gi)


def kernel(users, items, edge_index, user_emb, item_emb):
    e2 = edge_index.shape[1] // 2
    e2p = -(-e2 // (NSUB * CHUNK * 8)) * (NSUB * CHUNK * 8)

    u = edge_index[0, :e2]
    it = edge_index[1, :e2] - NU
    pad = jnp.full((e2p - e2,), PAD_ROW, i32)
    u2d = jnp.concatenate([u, pad]).reshape(e2p // CHUNK, CHUNK)
    i2d = jnp.concatenate([it, pad]).reshape(e2p // CHUNK, CHUNK)

    onesD = jnp.ones((CHUNK, D), f32)
    zrows = jnp.zeros((RB, D), f32)
    uep = jnp.pad(user_emb, ((0, NP - NU), (0, 0)))
    iep = jnp.pad(item_emb, ((0, NP - NI), (0, 0)))
    users2d = users.reshape(B // CHUNK, CHUNK)
    items2d = items.reshape(B // CHUNK, CHUNK)

    degu, degi = _deg(u2d, i2d, onesD, zrows)
    y0u, y0i, dinvu, dinvi = _prep(uep, iep, degu, degi)
    y1u, y1i = _layer(u2d, i2d, y0u, y0i, dinvu, dinvi, zrows)
    y2u, y2i = _layer(u2d, i2d, y1u, y1i, dinvu, dinvi, zrows)
    y3u, y3i = _layer(u2d, i2d, y2u, y2i, dinvu, dinvi, zrows)
    zu, zi = _comb(y0u, y1u, y2u, y3u, y0i, y1i, y2i, y3i, degu, degi)
    gu, gi = _bgather(zu, zi, users2d, items2d)
    return _gamma(gu, gi)[:, 0]


# speedup vs baseline: 14.9023x; 1.1006x over previous
import functools

import jax
import jax.numpy as jnp
from jax import lax
from jax.experimental import pallas as pl
from jax.experimental.pallas import tpu as pltpu
from jax.experimental.pallas import tpu_sc as plsc

NU = 5000
NI = 5000
D = 128
B = 4096
NP = 5120
PAD_ROW = NP - 1
NSUB = 16
CHUNK = 128
RB = 64
NBUF = 3

f32 = jnp.float32
i32 = jnp.int32

_mesh = plsc.VectorSubcoreMesh(core_axis_name="c", subcore_axis_name="s")


def _sds(shape, dtype):
    return jax.ShapeDtypeStruct(shape, dtype)


def _deg_body(cps, uidx_hbm, iidx_hbm, ones_hbm, zrows_hbm,
              degu_hbm, degi_hbm, idx_v, ones_v, sem, acc_sh):
    core = lax.axis_index("c")
    sid = lax.axis_index("s")
    rps = NP // NSUB

    @pl.loop(0, rps // RB)
    def _(j):
        pltpu.sync_copy(zrows_hbm, acc_sh.at[pl.ds(sid * rps + j * RB, RB)])

    pltpu.sync_copy(ones_hbm, ones_v)
    plsc.subcore_barrier()

    def _count(idx2d_hbm, deg_hbm):
        pltpu.sync_copy(idx2d_hbm.at[pl.ds(sid * cps, cps)], idx_v)

        @pl.loop(0, cps)
        def _(k):
            pltpu.sync_copy(ones_v, acc_sh.at[idx_v.at[k]], add=True)

        plsc.subcore_barrier()
        pltpu.sync_copy(acc_sh.at[pl.ds(sid * rps, rps)],
                        deg_hbm.at[pl.ds(sid * rps, rps)])

    @pl.when(core == 0)
    def _():
        _count(uidx_hbm, degu_hbm)

    @pl.when(core == 1)
    def _():
        _count(iidx_hbm, degi_hbm)


def _deg(u2d, i2d, onesD, zrows):
    cps = u2d.shape[0] // NSUB
    body = functools.partial(_deg_body, cps)
    k = pl.kernel(
        body,
        mesh=_mesh,
        out_type=[_sds((NP, D), f32), _sds((NP, D), f32)],
        scratch_types=[
            pltpu.VMEM((cps, CHUNK), i32),
            pltpu.VMEM((CHUNK, D), f32),
            pltpu.SemaphoreType.DMA,
            pltpu.VMEM_SHARED((NP, D), f32),
        ],
    )
    return k(u2d, i2d, onesD, zrows)


def _layer_body(cps, uidx_hbm, iidx_hbm, yu_hbm, yi_hbm, dinvu_hbm, dinvi_hbm,
                zrows_hbm, you_hbm, yoi_hbm,
                idxg0_v, idxs0_v, idxg1_v, idxs1_v, rows_a, rows_b, dinv_v,
                sem_a, sem_b, sem_i0, sem_i1, acc_sh, ysrc_sh):
    core = lax.axis_index("c")
    sid = lax.axis_index("s")
    rps = NP // NSUB
    nblk = 5
    hc = cps // nblk

    @pl.loop(0, CHUNK)
    def _(r):
        for q in range(D // 16):
            rows_a[r, pl.ds(q * 16, 16)] = jnp.full((16,), 0.0, f32)

    for j, sz in ((0, CHUNK), (1, CHUNK), (2, RB)):
        pltpu.sync_copy(rows_a.at[pl.ds(0, sz)],
                        acc_sh.at[pl.ds(sid * rps + j * CHUNK, sz)])

    @pl.when(core == 0)
    def _():
        pltpu.sync_copy(yi_hbm.at[pl.ds(sid * rps, rps)],
                        ysrc_sh.at[pl.ds(sid * rps, rps)])

    @pl.when(core == 1)
    def _():
        pltpu.sync_copy(yu_hbm.at[pl.ds(sid * rps, rps)],
                        ysrc_sh.at[pl.ds(sid * rps, rps)])

    plsc.subcore_barrier()

    def _side(idxg2d_hbm, idxs2d_hbm, dinv_hbm, yout_hbm):
        pairs = [(idxg0_v, idxs0_v, sem_i0), (idxg1_v, idxs1_v, sem_i1)]
        gbufs = [(rows_a, sem_a), (rows_b, sem_b)]

        def idx_load(h, sync):
            gv, sv, si = pairs[h % 2]
            base = sid * cps + h * hc
            if sync:
                pltpu.sync_copy(idxg2d_hbm.at[pl.ds(base, hc)], gv)
                pltpu.sync_copy(idxs2d_hbm.at[pl.ds(base, hc)], sv)
            else:
                pltpu.async_copy(idxg2d_hbm.at[pl.ds(base, hc)], gv, si)
                pltpu.async_copy(idxs2d_hbm.at[pl.ds(base, hc)], sv, si)

        def idx_wait(h):
            gv, sv, si = pairs[h % 2]
            base = sid * cps + h * hc
            pltpu.make_async_copy(idxg2d_hbm.at[pl.ds(base, hc)], gv,
                                  si).wait()
            pltpu.make_async_copy(idxs2d_hbm.at[pl.ds(base, hc)], sv,
                                  si).wait()

        def issue(g):
            h, k = divmod(g, hc)
            rv, sv = gbufs[g % 2]
            pltpu.async_copy(ysrc_sh.at[pairs[h % 2][0].at[k]], rv, sv)

        def drain_and_add(g):
            h, k = divmod(g, hc)
            rv, sv = gbufs[g % 2]
            pltpu.make_async_copy(ysrc_sh.at[pairs[h % 2][0].at[k]], rv,
                                  sv).wait()
            pltpu.sync_copy(rv, acc_sh.at[pairs[h % 2][1].at[k]], add=True)

        idx_load(0, sync=True)
        idx_load(1, sync=False)
        issue(0)
        issue(1)
        for g in range(cps):
            h, k = divmod(g, hc)
            if h + 1 < nblk and k == hc - 2:
                idx_wait(h + 1)
            drain_and_add(g)
            if g + 2 < cps:
                issue(g + 2)
            if k == hc - 1 and h + 2 < nblk:
                idx_load(h + 2, sync=False)

        plsc.subcore_barrier()

        @pl.loop(0, rps // RB)
        def _(j):
            r0 = sid * rps + j * RB
            pltpu.sync_copy(acc_sh.at[pl.ds(r0, RB)], rows_a.at[pl.ds(0, RB)])
            pltpu.sync_copy(dinv_hbm.at[pl.ds(r0, RB)], dinv_v)

            @pl.loop(0, RB)
            def _(r):
                dv = dinv_v[r, :]
                for q in range(D // 16):
                    sl = pl.ds(q * 16, 16)
                    rows_a[r, sl] = rows_a[r, sl] * dv

            pltpu.sync_copy(rows_a.at[pl.ds(0, RB)], yout_hbm.at[pl.ds(r0, RB)])

    @pl.when(core == 0)
    def _():
        _side(iidx_hbm, uidx_hbm, dinvu_hbm, you_hbm)

    @pl.when(core == 1)
    def _():
        _side(uidx_hbm, iidx_hbm, dinvi_hbm, yoi_hbm)


def _layer(u2d, i2d, yu, yi, dinvu, dinvi, zrows):
    cps = u2d.shape[0] // NSUB
    body = functools.partial(_layer_body, cps)
    k = pl.kernel(
        body,
        mesh=_mesh,
        out_type=[_sds((NP, D), f32), _sds((NP, D), f32)],
        scratch_types=[
            pltpu.VMEM((cps // 5, CHUNK), i32),
            pltpu.VMEM((cps // 5, CHUNK), i32),
            pltpu.VMEM((cps // 5, CHUNK), i32),
            pltpu.VMEM((cps // 5, CHUNK), i32),
            pltpu.VMEM((CHUNK, D), f32),
            pltpu.VMEM((CHUNK, D), f32),
            pltpu.VMEM((RB, 16), f32),
            pltpu.SemaphoreType.DMA,
            pltpu.SemaphoreType.DMA,
            pltpu.SemaphoreType.DMA,
            pltpu.SemaphoreType.DMA,
            pltpu.VMEM_SHARED((NP, D), f32),
            pltpu.VMEM_SHARED((NP, D), f32),
        ],
    )
    return k(u2d, i2d, yu, yi, dinvu, dinvi, zrows)


def _bgather_body(rows_per_sub, zu_hbm, zi_hbm, users_hbm, items_hbm,
                  gu_hbm, gi_hbm, idx_v, rows_v, sem):
    core = lax.axis_index("c")
    sid = lax.axis_index("s")
    nck = rows_per_sub // CHUNK

    def _side(idx2d_hbm, z_hbm, g_hbm):
        pltpu.sync_copy(idx2d_hbm, idx_v)

        @pl.loop(0, nck)
        def _(k):
            pltpu.sync_copy(z_hbm.at[idx_v.at[sid * nck + k]], rows_v)
            pltpu.sync_copy(rows_v,
                            g_hbm.at[pl.ds((sid * nck + k) * CHUNK, CHUNK)])

    @pl.when(core == 0)
    def _():
        _side(users_hbm, zu_hbm, gu_hbm)

    @pl.when(core == 1)
    def _():
        _side(items_hbm, zi_hbm, gi_hbm)


def _bgather(zu, zi, users2d, items2d):
    rows_per_sub = B // NSUB
    body = functools.partial(_bgather_body, rows_per_sub)
    k = pl.kernel(
        body,
        mesh=_mesh,
        out_type=[_sds((B, D), f32), _sds((B, D), f32)],
        scratch_types=[
            pltpu.VMEM((B // CHUNK, CHUNK), i32),
            pltpu.VMEM((CHUNK, D), f32),
            pltpu.SemaphoreType.DMA,
        ],
    )
    return k(zu, zi, users2d, items2d)


def _prep_tc_body(ue_ref, ie_ref, degu_ref, degi_ref,
                  y0u_ref, y0i_ref, dinvu_ref, dinvi_ref):
    du = jnp.maximum(degu_ref[:, 0:1], 1.0)
    di = jnp.maximum(degi_ref[:, 0:1], 1.0)
    y0u_ref[...] = ue_ref[...] * lax.rsqrt(du)
    y0i_ref[...] = ie_ref[...] * lax.rsqrt(di)
    dinvu_ref[...] = jnp.broadcast_to(1.0 / du, (NP, 16))
    dinvi_ref[...] = jnp.broadcast_to(1.0 / di, (NP, 16))


def _prep(uep, iep, degu, degi):
    return pl.pallas_call(
        _prep_tc_body,
        out_shape=[_sds((NP, D), f32), _sds((NP, D), f32),
                   _sds((NP, 16), f32), _sds((NP, 16), f32)],
    )(uep, iep, degu, degi)


def _comb_tc_body(y0u, y1u, y2u, y3u, y0i, y1i, y2i, y3i, degu, degi,
                  zu_ref, zi_ref):
    su = jnp.sqrt(jnp.maximum(degu[:, 0:1], 1.0)) * 0.25
    si = jnp.sqrt(jnp.maximum(degi[:, 0:1], 1.0)) * 0.25
    zu_ref[...] = (y0u[...] + y1u[...] + y2u[...] + y3u[...]) * su
    zi_ref[...] = (y0i[...] + y1i[...] + y2i[...] + y3i[...]) * si


def _comb(y0u, y1u, y2u, y3u, y0i, y1i, y2i, y3i, degu, degi):
    return pl.pallas_call(
        _comb_tc_body,
        out_shape=[_sds((NP, D), f32), _sds((NP, D), f32)],
    )(y0u, y1u, y2u, y3u, y0i, y1i, y2i, y3i, degu, degi)


def _gamma_tc_body(gu_ref, gi_ref, out_ref):
    out_ref[...] = jnp.sum(gu_ref[...] * gi_ref[...], axis=1, keepdims=True)


def _gamma(gu, gi):
    return pl.pallas_call(
        _gamma_tc_body,
        out_shape=_sds((B, 1), f32),
    )(gu, gi)


def kernel(users, items, edge_index, user_emb, item_emb):
    e2 = edge_index.shape[1] // 2
    e2p = -(-e2 // (NSUB * CHUNK * 8)) * (NSUB * CHUNK * 8)

    u = edge_index[0, :e2]
    it = edge_index[1, :e2] - NU
    pad = jnp.full((e2p - e2,), PAD_ROW, i32)
    u2d = jnp.concatenate([u, pad]).reshape(e2p // CHUNK, CHUNK)
    i2d = jnp.concatenate([it, pad]).reshape(e2p // CHUNK, CHUNK)

    onesD = jnp.ones((CHUNK, D), f32)
    zrows = jnp.zeros((RB, D), f32)
    uep = jnp.pad(user_emb, ((0, NP - NU), (0, 0)))
    iep = jnp.pad(item_emb, ((0, NP - NI), (0, 0)))
    users2d = users.reshape(B // CHUNK, CHUNK)
    items2d = items.reshape(B // CHUNK, CHUNK)

    degu, degi = _deg(u2d, i2d, onesD, zrows)
    y0u, y0i, dinvu, dinvi = _prep(uep, iep, degu, degi)
    y1u, y1i = _layer(u2d, i2d, y0u, y0i, dinvu, dinvi, zrows)
    y2u, y2i = _layer(u2d, i2d, y1u, y1i, dinvu, dinvi, zrows)
    y3u, y3i = _layer(u2d, i2d, y2u, y2i, dinvu, dinvi, zrows)
    zu, zi = _comb(y0u, y1u, y2u, y3u, y0i, y1i, y2i, y3i, degu, degi)
    gu, gi = _bgather(zu, zi, users2d, items2d)
    return _gamma(gu, gi)[:, 0]


# speedup vs baseline: 14.9433x; 1.0028x over previous
import functools

import jax
import jax.numpy as jnp
from jax import lax
from jax.experimental import pallas as pl
from jax.experimental.pallas import tpu as pltpu
from jax.experimental.pallas import tpu_sc as plsc

NU = 5000
NI = 5000
D = 128
B = 4096
NP = 5120
PAD_ROW = NP - 1
NSUB = 16
CHUNK = 128
RB = 64
NBUF = 3

f32 = jnp.float32
i32 = jnp.int32

_mesh = plsc.VectorSubcoreMesh(core_axis_name="c", subcore_axis_name="s")


def _sds(shape, dtype):
    return jax.ShapeDtypeStruct(shape, dtype)


def _deg_body(cps, uidx_hbm, iidx_hbm, ones_hbm, zrows_hbm,
              degu_hbm, degi_hbm, idx_v, ones_v, sem, acc_sh):
    core = lax.axis_index("c")
    sid = lax.axis_index("s")
    rps = NP // NSUB

    @pl.loop(0, rps // RB)
    def _(j):
        pltpu.sync_copy(zrows_hbm, acc_sh.at[pl.ds(sid * rps + j * RB, RB)])

    pltpu.sync_copy(ones_hbm, ones_v)
    plsc.subcore_barrier()

    def _count(idx2d_hbm, deg_hbm):
        pltpu.sync_copy(idx2d_hbm.at[pl.ds(sid * cps, cps)], idx_v)

        @pl.loop(0, cps)
        def _(k):
            pltpu.async_copy(ones_v, acc_sh.at[idx_v.at[k]], sem, add=True)

        @pl.loop(0, cps)
        def _(k):
            pltpu.make_async_copy(ones_v, acc_sh.at[idx_v.at[k]], sem).wait()

        plsc.subcore_barrier()
        pltpu.sync_copy(acc_sh.at[pl.ds(sid * rps, rps)],
                        deg_hbm.at[pl.ds(sid * rps, rps)])

    @pl.when(core == 0)
    def _():
        _count(uidx_hbm, degu_hbm)

    @pl.when(core == 1)
    def _():
        _count(iidx_hbm, degi_hbm)


def _deg(u2d, i2d, onesD, zrows):
    cps = u2d.shape[0] // NSUB
    body = functools.partial(_deg_body, cps)
    k = pl.kernel(
        body,
        mesh=_mesh,
        out_type=[_sds((NP, D), f32), _sds((NP, D), f32)],
        scratch_types=[
            pltpu.VMEM((cps, CHUNK), i32),
            pltpu.VMEM((CHUNK, D), f32),
            pltpu.SemaphoreType.DMA,
            pltpu.VMEM_SHARED((NP, D), f32),
        ],
    )
    return k(u2d, i2d, onesD, zrows)


def _layer_body(cps, uidx_hbm, iidx_hbm, yu_hbm, yi_hbm, dinvu_hbm, dinvi_hbm,
                zrows_hbm, you_hbm, yoi_hbm,
                idxg0_v, idxs0_v, idxg1_v, idxs1_v, rows_a, rows_b, dinv_v,
                sem_a, sem_b, sem_i0, sem_i1, acc_sh, ysrc_sh):
    core = lax.axis_index("c")
    sid = lax.axis_index("s")
    rps = NP // NSUB
    nblk = 5
    hc = cps // nblk

    @pl.loop(0, CHUNK)
    def _(r):
        for q in range(D // 16):
            rows_a[r, pl.ds(q * 16, 16)] = jnp.full((16,), 0.0, f32)

    for j, sz in ((0, CHUNK), (1, CHUNK), (2, RB)):
        pltpu.sync_copy(rows_a.at[pl.ds(0, sz)],
                        acc_sh.at[pl.ds(sid * rps + j * CHUNK, sz)])

    @pl.when(core == 0)
    def _():
        pltpu.sync_copy(yi_hbm.at[pl.ds(sid * rps, rps)],
                        ysrc_sh.at[pl.ds(sid * rps, rps)])

    @pl.when(core == 1)
    def _():
        pltpu.sync_copy(yu_hbm.at[pl.ds(sid * rps, rps)],
                        ysrc_sh.at[pl.ds(sid * rps, rps)])

    plsc.subcore_barrier()

    def _side(idxg2d_hbm, idxs2d_hbm, dinv_hbm, yout_hbm):
        pairs = [(idxg0_v, idxs0_v, sem_i0), (idxg1_v, idxs1_v, sem_i1)]
        gbufs = [(rows_a, sem_a), (rows_b, sem_b)]

        def idx_load(h, sync):
            gv, sv, si = pairs[h % 2]
            base = sid * cps + h * hc
            if sync:
                pltpu.sync_copy(idxg2d_hbm.at[pl.ds(base, hc)], gv)
                pltpu.sync_copy(idxs2d_hbm.at[pl.ds(base, hc)], sv)
            else:
                pltpu.async_copy(idxg2d_hbm.at[pl.ds(base, hc)], gv, si)
                pltpu.async_copy(idxs2d_hbm.at[pl.ds(base, hc)], sv, si)

        def idx_wait(h):
            gv, sv, si = pairs[h % 2]
            base = sid * cps + h * hc
            pltpu.make_async_copy(idxg2d_hbm.at[pl.ds(base, hc)], gv,
                                  si).wait()
            pltpu.make_async_copy(idxs2d_hbm.at[pl.ds(base, hc)], sv,
                                  si).wait()

        def issue(g):
            h, k = divmod(g, hc)
            rv, sv = gbufs[g % 2]
            pltpu.async_copy(ysrc_sh.at[pairs[h % 2][0].at[k]], rv, sv)

        def drain_and_add(g):
            h, k = divmod(g, hc)
            rv, sv = gbufs[g % 2]
            pltpu.make_async_copy(ysrc_sh.at[pairs[h % 2][0].at[k]], rv,
                                  sv).wait()
            pltpu.sync_copy(rv, acc_sh.at[pairs[h % 2][1].at[k]], add=True)

        idx_load(0, sync=True)
        idx_load(1, sync=False)
        issue(0)
        issue(1)
        for g in range(cps):
            h, k = divmod(g, hc)
            if h + 1 < nblk and k == hc - 2:
                idx_wait(h + 1)
            drain_and_add(g)
            if g + 2 < cps:
                issue(g + 2)
            if k == hc - 1 and h + 2 < nblk:
                idx_load(h + 2, sync=False)

        plsc.subcore_barrier()

        @pl.loop(0, rps // RB)
        def _(j):
            r0 = sid * rps + j * RB
            pltpu.sync_copy(acc_sh.at[pl.ds(r0, RB)], rows_a.at[pl.ds(0, RB)])
            pltpu.sync_copy(dinv_hbm.at[pl.ds(r0, RB)], dinv_v)

            @pl.loop(0, RB)
            def _(r):
                dv = dinv_v[r, :]
                for q in range(D // 16):
                    sl = pl.ds(q * 16, 16)
                    rows_a[r, sl] = rows_a[r, sl] * dv

            pltpu.sync_copy(rows_a.at[pl.ds(0, RB)], yout_hbm.at[pl.ds(r0, RB)])

    @pl.when(core == 0)
    def _():
        _side(iidx_hbm, uidx_hbm, dinvu_hbm, you_hbm)

    @pl.when(core == 1)
    def _():
        _side(uidx_hbm, iidx_hbm, dinvi_hbm, yoi_hbm)


def _layer(u2d, i2d, yu, yi, dinvu, dinvi, zrows):
    cps = u2d.shape[0] // NSUB
    body = functools.partial(_layer_body, cps)
    k = pl.kernel(
        body,
        mesh=_mesh,
        out_type=[_sds((NP, D), f32), _sds((NP, D), f32)],
        scratch_types=[
            pltpu.VMEM((cps // 5, CHUNK), i32),
            pltpu.VMEM((cps // 5, CHUNK), i32),
            pltpu.VMEM((cps // 5, CHUNK), i32),
            pltpu.VMEM((cps // 5, CHUNK), i32),
            pltpu.VMEM((CHUNK, D), f32),
            pltpu.VMEM((CHUNK, D), f32),
            pltpu.VMEM((RB, 16), f32),
            pltpu.SemaphoreType.DMA,
            pltpu.SemaphoreType.DMA,
            pltpu.SemaphoreType.DMA,
            pltpu.SemaphoreType.DMA,
            pltpu.VMEM_SHARED((NP, D), f32),
            pltpu.VMEM_SHARED((NP, D), f32),
        ],
    )
    return k(u2d, i2d, yu, yi, dinvu, dinvi, zrows)


def _bgather_body(rows_per_sub, zu_hbm, zi_hbm, users_hbm, items_hbm,
                  gu_hbm, gi_hbm, idx_v, rows_v, sem):
    core = lax.axis_index("c")
    sid = lax.axis_index("s")
    nck = rows_per_sub // CHUNK

    def _side(idx2d_hbm, z_hbm, g_hbm):
        pltpu.sync_copy(idx2d_hbm, idx_v)

        @pl.loop(0, nck)
        def _(k):
            pltpu.sync_copy(z_hbm.at[idx_v.at[sid * nck + k]], rows_v)
            pltpu.sync_copy(rows_v,
                            g_hbm.at[pl.ds((sid * nck + k) * CHUNK, CHUNK)])

    @pl.when(core == 0)
    def _():
        _side(users_hbm, zu_hbm, gu_hbm)

    @pl.when(core == 1)
    def _():
        _side(items_hbm, zi_hbm, gi_hbm)


def _bgather(zu, zi, users2d, items2d):
    rows_per_sub = B // NSUB
    body = functools.partial(_bgather_body, rows_per_sub)
    k = pl.kernel(
        body,
        mesh=_mesh,
        out_type=[_sds((B, D), f32), _sds((B, D), f32)],
        scratch_types=[
            pltpu.VMEM((B // CHUNK, CHUNK), i32),
            pltpu.VMEM((CHUNK, D), f32),
            pltpu.SemaphoreType.DMA,
        ],
    )
    return k(zu, zi, users2d, items2d)


def _prep_tc_body(ue_ref, ie_ref, degu_ref, degi_ref,
                  y0u_ref, y0i_ref, dinvu_ref, dinvi_ref):
    du = jnp.maximum(degu_ref[:, 0:1], 1.0)
    di = jnp.maximum(degi_ref[:, 0:1], 1.0)
    y0u_ref[...] = ue_ref[...] * lax.rsqrt(du)
    y0i_ref[...] = ie_ref[...] * lax.rsqrt(di)
    dinvu_ref[...] = jnp.broadcast_to(1.0 / du, (NP, 16))
    dinvi_ref[...] = jnp.broadcast_to(1.0 / di, (NP, 16))


def _prep(uep, iep, degu, degi):
    return pl.pallas_call(
        _prep_tc_body,
        out_shape=[_sds((NP, D), f32), _sds((NP, D), f32),
                   _sds((NP, 16), f32), _sds((NP, 16), f32)],
    )(uep, iep, degu, degi)


def _comb_tc_body(y0u, y1u, y2u, y3u, y0i, y1i, y2i, y3i, degu, degi,
                  zu_ref, zi_ref):
    su = jnp.sqrt(jnp.maximum(degu[:, 0:1], 1.0)) * 0.25
    si = jnp.sqrt(jnp.maximum(degi[:, 0:1], 1.0)) * 0.25
    zu_ref[...] = (y0u[...] + y1u[...] + y2u[...] + y3u[...]) * su
    zi_ref[...] = (y0i[...] + y1i[...] + y2i[...] + y3i[...]) * si


def _comb(y0u, y1u, y2u, y3u, y0i, y1i, y2i, y3i, degu, degi):
    return pl.pallas_call(
        _comb_tc_body,
        out_shape=[_sds((NP, D), f32), _sds((NP, D), f32)],
    )(y0u, y1u, y2u, y3u, y0i, y1i, y2i, y3i, degu, degi)


def _gamma_tc_body(gu_ref, gi_ref, out_ref):
    out_ref[...] = jnp.sum(gu_ref[...] * gi_ref[...], axis=1, keepdims=True)


def _gamma(gu, gi):
    return pl.pallas_call(
        _gamma_tc_body,
        out_shape=_sds((B, 1), f32),
    )(gu, gi)


def kernel(users, items, edge_index, user_emb, item_emb):
    e2 = edge_index.shape[1] // 2
    e2p = -(-e2 // (NSUB * CHUNK * 8)) * (NSUB * CHUNK * 8)

    u = edge_index[0, :e2]
    it = edge_index[1, :e2] - NU
    pad = jnp.full((e2p - e2,), PAD_ROW, i32)
    u2d = jnp.concatenate([u, pad]).reshape(e2p // CHUNK, CHUNK)
    i2d = jnp.concatenate([it, pad]).reshape(e2p // CHUNK, CHUNK)

    onesD = jnp.ones((CHUNK, D), f32)
    zrows = jnp.zeros((RB, D), f32)
    uep = jnp.pad(user_emb, ((0, NP - NU), (0, 0)))
    iep = jnp.pad(item_emb, ((0, NP - NI), (0, 0)))
    users2d = users.reshape(B // CHUNK, CHUNK)
    items2d = items.reshape(B // CHUNK, CHUNK)

    degu, degi = _deg(u2d, i2d, onesD, zrows)
    y0u, y0i, dinvu, dinvi = _prep(uep, iep, degu, degi)
    y1u, y1i = _layer(u2d, i2d, y0u, y0i, dinvu, dinvi, zrows)
    y2u, y2i = _layer(u2d, i2d, y1u, y1i, dinvu, dinvi, zrows)
    y3u, y3i = _layer(u2d, i2d, y2u, y2i, dinvu, dinvi, zrows)
    zu, zi = _comb(y0u, y1u, y2u, y3u, y0i, y1i, y2i, y3i, degu, degi)
    gu, gi = _bgather(zu, zi, users2d, items2d)
    return _gamma(gu, gi)[:, 0]


# speedup vs baseline: 16.5552x; 1.1079x over previous
import functools

import jax
import jax.numpy as jnp
from jax import lax
from jax.experimental import pallas as pl
from jax.experimental.pallas import tpu as pltpu
from jax.experimental.pallas import tpu_sc as plsc

NU = 5000
NI = 5000
D = 128
B = 4096
NP = 5120
PAD_ROW = NP - 1
NSUB = 16
CHUNK = 128
RB = 64
NBUF = 3

f32 = jnp.float32
i32 = jnp.int32

_mesh = plsc.VectorSubcoreMesh(core_axis_name="c", subcore_axis_name="s")


def _sds(shape, dtype):
    return jax.ShapeDtypeStruct(shape, dtype)


_cp_sc = pltpu.CompilerParams()
if "needs_layout_passes" in pltpu.CompilerParams.__dataclass_fields__:
    import dataclasses as _dc
    _cp_sc = _dc.replace(_cp_sc, needs_layout_passes=False)


def _deg_body(cps, uidx_hbm, iidx_hbm, degu_hbm, degi_hbm,
              idx_v, hist_v, red_l, sem, red_sh):
    core = lax.axis_index("c")
    sid = lax.axis_index("s")
    rps = NP // NSUB
    ones = jnp.full((16,), 1.0, f32)

    @pl.loop(0, NP, step=16)
    def _(j):
        hist_v[pl.ds(j, 16)] = jnp.full((16,), 0.0, f32)

    def _count(idx2d_hbm, deg_hbm):
        pltpu.sync_copy(idx2d_hbm.at[pl.ds(sid * cps, cps)], idx_v)

        @pl.loop(0, cps)
        def _(k):
            for q in range(CHUNK // 16):
                plsc.addupdate_scatter(hist_v,
                                       [idx_v[k, pl.ds(q * 16, 16)]], ones)

        pltpu.sync_copy(hist_v, red_sh.at[pl.ds(sid * NP, NP)])
        plsc.subcore_barrier()
        for t in range(NSUB):
            pltpu.sync_copy(red_sh.at[pl.ds(t * NP + sid * rps, rps)],
                            red_l.at[pl.ds(t * rps, rps)])

        @pl.loop(0, rps, step=16)
        def _(j):
            v = red_l[pl.ds(j, 16)]
            for t in range(1, NSUB):
                v = v + red_l[pl.ds(t * rps + j, 16)]
            hist_v[pl.ds(j, 16)] = v

        pltpu.sync_copy(hist_v.at[pl.ds(0, rps)],
                        deg_hbm.at[pl.ds(sid * rps, rps)])

    @pl.when(core == 0)
    def _():
        _count(uidx_hbm, degu_hbm)

    @pl.when(core == 1)
    def _():
        _count(iidx_hbm, degi_hbm)


def _deg(u2d, i2d):
    cps = u2d.shape[0] // NSUB
    rps = NP // NSUB
    body = functools.partial(_deg_body, cps)
    k = pl.kernel(
        body,
        mesh=_mesh,
        out_type=[_sds((NP,), f32), _sds((NP,), f32)],
        compiler_params=_cp_sc,
        scratch_types=[
            pltpu.VMEM((cps, CHUNK), i32),
            pltpu.VMEM((NP,), f32),
            pltpu.VMEM((NSUB * rps,), f32),
            pltpu.SemaphoreType.DMA,
            pltpu.VMEM_SHARED((NSUB * NP,), f32),
        ],
    )
    return k(u2d, i2d)


def _layer_body(cps, uidx_hbm, iidx_hbm, yu_hbm, yi_hbm, dinvu_hbm, dinvi_hbm,
                you_hbm, yoi_hbm,
                idxg0_v, idxs0_v, idxg1_v, idxs1_v, rows_a, rows_b, dinv_v,
                sem_a, sem_b, sem_i0, sem_i1, acc_sh, ysrc_sh):
    core = lax.axis_index("c")
    sid = lax.axis_index("s")
    rps = NP // NSUB
    nblk = 5
    hc = cps // nblk

    @pl.loop(0, CHUNK)
    def _(r):
        for q in range(D // 16):
            rows_a[r, pl.ds(q * 16, 16)] = jnp.full((16,), 0.0, f32)

    for j, sz in ((0, CHUNK), (1, CHUNK), (2, RB)):
        pltpu.sync_copy(rows_a.at[pl.ds(0, sz)],
                        acc_sh.at[pl.ds(sid * rps + j * CHUNK, sz)])

    @pl.when(core == 0)
    def _():
        pltpu.sync_copy(yi_hbm.at[pl.ds(sid * rps, rps)],
                        ysrc_sh.at[pl.ds(sid * rps, rps)])

    @pl.when(core == 1)
    def _():
        pltpu.sync_copy(yu_hbm.at[pl.ds(sid * rps, rps)],
                        ysrc_sh.at[pl.ds(sid * rps, rps)])

    plsc.subcore_barrier()

    def _side(idxg2d_hbm, idxs2d_hbm, dinv_hbm, yout_hbm):
        pairs = [(idxg0_v, idxs0_v, sem_i0), (idxg1_v, idxs1_v, sem_i1)]
        gbufs = [(rows_a, sem_a), (rows_b, sem_b)]

        def idx_load(h, sync):
            gv, sv, si = pairs[h % 2]
            base = sid * cps + h * hc
            if sync:
                pltpu.sync_copy(idxg2d_hbm.at[pl.ds(base, hc)], gv)
                pltpu.sync_copy(idxs2d_hbm.at[pl.ds(base, hc)], sv)
            else:
                pltpu.async_copy(idxg2d_hbm.at[pl.ds(base, hc)], gv, si)
                pltpu.async_copy(idxs2d_hbm.at[pl.ds(base, hc)], sv, si)

        def idx_wait(h):
            gv, sv, si = pairs[h % 2]
            base = sid * cps + h * hc
            pltpu.make_async_copy(idxg2d_hbm.at[pl.ds(base, hc)], gv,
                                  si).wait()
            pltpu.make_async_copy(idxs2d_hbm.at[pl.ds(base, hc)], sv,
                                  si).wait()

        def issue(g):
            h, k = divmod(g, hc)
            rv, sv = gbufs[g % 2]
            pltpu.async_copy(ysrc_sh.at[pairs[h % 2][0].at[k]], rv, sv)

        def drain_and_add(g):
            h, k = divmod(g, hc)
            rv, sv = gbufs[g % 2]
            pltpu.make_async_copy(ysrc_sh.at[pairs[h % 2][0].at[k]], rv,
                                  sv).wait()
            pltpu.sync_copy(rv, acc_sh.at[pairs[h % 2][1].at[k]], add=True)

        idx_load(0, sync=True)
        idx_load(1, sync=False)
        issue(0)
        issue(1)
        for g in range(cps):
            h, k = divmod(g, hc)
            if h + 1 < nblk and k == hc - 2:
                idx_wait(h + 1)
            drain_and_add(g)
            if g + 2 < cps:
                issue(g + 2)
            if k == hc - 1 and h + 2 < nblk:
                idx_load(h + 2, sync=False)

        plsc.subcore_barrier()

        @pl.loop(0, rps // RB)
        def _(j):
            r0 = sid * rps + j * RB
            pltpu.sync_copy(acc_sh.at[pl.ds(r0, RB)], rows_a.at[pl.ds(0, RB)])
            pltpu.sync_copy(dinv_hbm.at[pl.ds(r0, RB)], dinv_v)

            @pl.loop(0, RB)
            def _(r):
                dv = dinv_v[r, :]
                for q in range(D // 16):
                    sl = pl.ds(q * 16, 16)
                    rows_a[r, sl] = rows_a[r, sl] * dv

            pltpu.sync_copy(rows_a.at[pl.ds(0, RB)], yout_hbm.at[pl.ds(r0, RB)])

    @pl.when(core == 0)
    def _():
        _side(iidx_hbm, uidx_hbm, dinvu_hbm, you_hbm)

    @pl.when(core == 1)
    def _():
        _side(uidx_hbm, iidx_hbm, dinvi_hbm, yoi_hbm)


def _layer(u2d, i2d, yu, yi, dinvu, dinvi):
    cps = u2d.shape[0] // NSUB
    body = functools.partial(_layer_body, cps)
    k = pl.kernel(
        body,
        mesh=_mesh,
        out_type=[_sds((NP, D), f32), _sds((NP, D), f32)],
        scratch_types=[
            pltpu.VMEM((cps // 5, CHUNK), i32),
            pltpu.VMEM((cps // 5, CHUNK), i32),
            pltpu.VMEM((cps // 5, CHUNK), i32),
            pltpu.VMEM((cps // 5, CHUNK), i32),
            pltpu.VMEM((CHUNK, D), f32),
            pltpu.VMEM((CHUNK, D), f32),
            pltpu.VMEM((RB, 16), f32),
            pltpu.SemaphoreType.DMA,
            pltpu.SemaphoreType.DMA,
            pltpu.SemaphoreType.DMA,
            pltpu.SemaphoreType.DMA,
            pltpu.VMEM_SHARED((NP, D), f32),
            pltpu.VMEM_SHARED((NP, D), f32),
        ],
    )
    return k(u2d, i2d, yu, yi, dinvu, dinvi)


def _bgather_body(rows_per_sub, zu_hbm, zi_hbm, users_hbm, items_hbm,
                  gu_hbm, gi_hbm, idx_v, rows_v, sem):
    core = lax.axis_index("c")
    sid = lax.axis_index("s")
    nck = rows_per_sub // CHUNK

    def _side(idx2d_hbm, z_hbm, g_hbm):
        pltpu.sync_copy(idx2d_hbm, idx_v)

        @pl.loop(0, nck)
        def _(k):
            pltpu.sync_copy(z_hbm.at[idx_v.at[sid * nck + k]], rows_v)
            pltpu.sync_copy(rows_v,
                            g_hbm.at[pl.ds((sid * nck + k) * CHUNK, CHUNK)])

    @pl.when(core == 0)
    def _():
        _side(users_hbm, zu_hbm, gu_hbm)

    @pl.when(core == 1)
    def _():
        _side(items_hbm, zi_hbm, gi_hbm)


def _bgather(zu, zi, users2d, items2d):
    rows_per_sub = B // NSUB
    body = functools.partial(_bgather_body, rows_per_sub)
    k = pl.kernel(
        body,
        mesh=_mesh,
        out_type=[_sds((B, D), f32), _sds((B, D), f32)],
        scratch_types=[
            pltpu.VMEM((B // CHUNK, CHUNK), i32),
            pltpu.VMEM((CHUNK, D), f32),
            pltpu.SemaphoreType.DMA,
        ],
    )
    return k(zu, zi, users2d, items2d)


def _prep_tc_body(ue_ref, ie_ref, degu_ref, degi_ref,
                  y0u_ref, y0i_ref, dinvu_ref, dinvi_ref):
    du = jnp.maximum(degu_ref[:, 0:1], 1.0)
    di = jnp.maximum(degi_ref[:, 0:1], 1.0)
    y0u_ref[...] = ue_ref[...] * lax.rsqrt(du)
    y0i_ref[...] = ie_ref[...] * lax.rsqrt(di)
    dinvu_ref[...] = jnp.broadcast_to(1.0 / du, (NP, 16))
    dinvi_ref[...] = jnp.broadcast_to(1.0 / di, (NP, 16))


def _prep(uep, iep, degu, degi):
    return pl.pallas_call(
        _prep_tc_body,
        out_shape=[_sds((NP, D), f32), _sds((NP, D), f32),
                   _sds((NP, 16), f32), _sds((NP, 16), f32)],
    )(uep, iep, degu, degi)


def _comb_tc_body(y0u, y1u, y2u, y3u, y0i, y1i, y2i, y3i, degu, degi,
                  zu_ref, zi_ref):
    su = jnp.sqrt(jnp.maximum(degu[:, 0:1], 1.0)) * 0.25
    si = jnp.sqrt(jnp.maximum(degi[:, 0:1], 1.0)) * 0.25
    zu_ref[...] = (y0u[...] + y1u[...] + y2u[...] + y3u[...]) * su
    zi_ref[...] = (y0i[...] + y1i[...] + y2i[...] + y3i[...]) * si


def _comb(y0u, y1u, y2u, y3u, y0i, y1i, y2i, y3i, degu, degi):
    return pl.pallas_call(
        _comb_tc_body,
        out_shape=[_sds((NP, D), f32), _sds((NP, D), f32)],
    )(y0u, y1u, y2u, y3u, y0i, y1i, y2i, y3i, degu, degi)


def _gamma_tc_body(gu_ref, gi_ref, out_ref):
    out_ref[...] = jnp.sum(gu_ref[...] * gi_ref[...], axis=1, keepdims=True)


def _gamma(gu, gi):
    return pl.pallas_call(
        _gamma_tc_body,
        out_shape=_sds((B, 1), f32),
    )(gu, gi)


def kernel(users, items, edge_index, user_emb, item_emb):
    e2 = edge_index.shape[1] // 2
    e2p = -(-e2 // (NSUB * CHUNK * 8)) * (NSUB * CHUNK * 8)

    u = edge_index[0, :e2]
    it = edge_index[1, :e2] - NU
    pad = jnp.full((e2p - e2,), PAD_ROW, i32)
    u2d = jnp.concatenate([u, pad]).reshape(e2p // CHUNK, CHUNK)
    i2d = jnp.concatenate([it, pad]).reshape(e2p // CHUNK, CHUNK)

    uep = jnp.pad(user_emb, ((0, NP - NU), (0, 0)))
    iep = jnp.pad(item_emb, ((0, NP - NI), (0, 0)))
    users2d = users.reshape(B // CHUNK, CHUNK)
    items2d = items.reshape(B // CHUNK, CHUNK)

    degu, degi = _deg(u2d, i2d)
    degu = degu.reshape(NP, 1)
    degi = degi.reshape(NP, 1)
    y0u, y0i, dinvu, dinvi = _prep(uep, iep, degu, degi)
    y1u, y1i = _layer(u2d, i2d, y0u, y0i, dinvu, dinvi)
    y2u, y2i = _layer(u2d, i2d, y1u, y1i, dinvu, dinvi)
    y3u, y3i = _layer(u2d, i2d, y2u, y2i, dinvu, dinvi)
    zu, zi = _comb(y0u, y1u, y2u, y3u, y0i, y1i, y2i, y3i, degu, degi)
    gu, gi = _bgather(zu, zi, users2d, items2d)
    return _gamma(gu, gi)[:, 0]
